# baseline (speedup 1.0000x reference)
import jax
import jax.numpy as jnp
from jax.experimental import pallas as pl

N_PTS = 4096
BLK = 256
KNN = 11
SWEEPS = 15
PIVOTS = ((0, 2), (1, 2), (0, 1))
S_SIGN = 1.0

_EIDX = {(0, 0): 0, (0, 1): 1, (0, 2): 2, (1, 1): 3, (1, 2): 4, (2, 2): 5}


def _e(i, j):
    return _EIDX[(i, j) if i <= j else (j, i)]


def _knn_cov_body(x_ref, xt_ref, idx_ref, cov_ref):
    x = x_ref[...]
    xt = xt_ref[...]
    sq_row = (xt[0:1, :] * xt[0:1, :] + xt[1:2, :] * xt[1:2, :]) \
        + xt[2:3, :] * xt[2:3, :]
    sq_blk = (x[:, 0:1] * x[:, 0:1] + x[:, 1:2] * x[:, 1:2]) \
        + x[:, 2:3] * x[:, 2:3]
    dot = jax.lax.dot_general(
        x, xt, (((1,), (0,)), ((), ())),
        preferred_element_type=jnp.float32)
    d2 = (sq_blk + sq_row) - 2.0 * dot

    iota = jax.lax.broadcasted_iota(jnp.int32, (BLK, N_PTS), 1)
    inf = jnp.float32(jnp.inf)
    big = jnp.int32(N_PTS)

    neigh = []
    for s in range(KNN):
        m = jnp.min(d2, axis=1, keepdims=True)
        hit = d2 == m
        idx = jnp.min(jnp.where(hit, iota, big), axis=1, keepdims=True)
        onehot = iota == idx
        idx_ref[:, s] = idx[:, 0]
        nb = jax.lax.dot_general(
            jnp.where(onehot, jnp.float32(1.0), jnp.float32(0.0)), xt,
            (((1,), (1,)), ((), ())),
            preferred_element_type=jnp.float32)
        neigh.append(nb)
        d2 = jnp.where(onehot, inf, d2)

    ssum = neigh[0]
    for s in range(1, KNN):
        ssum = ssum + neigh[s]
    mean = ssum / jnp.float32(KNN)
    cent = [nb - mean for nb in neigh]
    pairs = [(0, 0), (0, 1), (0, 2), (1, 1), (1, 2), (2, 2)]
    for e, (a, b) in enumerate(pairs):
        acc = cent[0][:, a:a + 1] * cent[0][:, b:b + 1]
        for s in range(1, KNN):
            acc = acc + cent[s][:, a:a + 1] * cent[s][:, b:b + 1]
        cov_ref[:, e] = (acc / jnp.float32(KNN - 1))[:, 0]


def _normals_body(cov_ref, xt_ref, out_ref):
    a = [cov_ref[e, :][None, :] for e in range(6)]
    one = jnp.ones((1, N_PTS), jnp.float32)
    zer = jnp.zeros((1, N_PTS), jnp.float32)
    v = [[one if i == j else zer for j in range(3)] for i in range(3)]

    for _ in range(SWEEPS):
        for (p, q) in PIVOTS:
            r = 3 - p - q
            app = a[_e(p, p)]
            aqq = a[_e(q, q)]
            apq = a[_e(p, q)]
            apr = a[_e(p, r)]
            aqr = a[_e(q, r)]
            iszero = apq == 0.0
            tau = (aqq - app) / (2.0 * apq)
            sq = jnp.sqrt(1.0 + tau * tau)
            t = jnp.where(tau >= 0.0, 1.0 / (tau + sq), 1.0 / (tau - sq))
            t = jnp.where(iszero, zer, t)
            c = 1.0 / jnp.sqrt(1.0 + t * t)
            s = jnp.float32(S_SIGN) * (t * c)
            a[_e(p, p)] = app - t * apq
            a[_e(q, q)] = aqq + t * apq
            a[_e(p, q)] = zer
            a[_e(p, r)] = c * apr - s * aqr
            a[_e(q, r)] = s * apr + c * aqr
            for i in range(3):
                vip, viq = v[i][p], v[i][q]
                v[i][p] = c * vip - s * viq
                v[i][q] = s * vip + c * viq

    w0, w1, w2 = a[0], a[3], a[5]
    b1 = w1 < w0
    bestw = jnp.where(b1, w1, w0)
    b2 = w2 < bestw
    n = [jnp.where(b2, v[i][2], jnp.where(b1, v[i][1], v[i][0]))
         for i in range(3)]
    nrm = jnp.sqrt(n[0] * n[0] + n[1] * n[1] + n[2] * n[2])
    n = [x / nrm for x in n]

    inv_n = jnp.float32(1.0 / N_PTS)
    lane = jax.lax.broadcasted_iota(jnp.int32, (1, 128), 1)
    acc = jnp.zeros((1, 128), jnp.float32)
    for ci in range(3):
        m = jnp.sum(xt_ref[ci, :][None, :], axis=1, keepdims=True) * inv_n
        acc = jnp.where(lane == ci, m, acc)
    for ci in range(3):
        m = jnp.sum(n[ci], axis=1, keepdims=True) * inv_n
        acc = jnp.where(lane == 3 + ci, m, acc)
    out_ref[...] = jnp.broadcast_to(acc, (8, 128))


def _sup_body(ai_ref, sdf_ref, xs_ref, ys_ref):
    ai = ai_ref[...]
    xs_ref[:, 0:3] = ai[:, 0:3]
    xs_ref[:, 3:4] = sdf_ref[:, 3:4]
    ys_ref[:, 0:1] = ai[:, 3:4]
    ys_ref[:, 1:2] = ai[:, 4:5] - 0.5
    ys_ref[:, 2:5] = ai[:, 5:8]


def kernel(array_internal, array_sdf, array_inlet, k):
    n_int = array_internal.shape[0]
    x_inlet = array_inlet[:, 0:3]
    xt = x_inlet.T

    _, cov6 = pl.pallas_call(
        _knn_cov_body,
        grid=(N_PTS // BLK,),
        in_specs=[
            pl.BlockSpec((BLK, 3), lambda i: (i, 0)),
            pl.BlockSpec((3, N_PTS), lambda i: (0, 0)),
        ],
        out_specs=[
            pl.BlockSpec((BLK, KNN), lambda i: (i, 0)),
            pl.BlockSpec((BLK, 6), lambda i: (i, 0)),
        ],
        out_shape=[
            jax.ShapeDtypeStruct((N_PTS, KNN), jnp.int32),
            jax.ShapeDtypeStruct((N_PTS, 6), jnp.float32),
        ],
    )(x_inlet, xt)

    out8 = pl.pallas_call(
        _normals_body,
        grid=(1,),
        in_specs=[
            pl.BlockSpec((6, N_PTS), lambda i: (0, 0)),
            pl.BlockSpec((3, N_PTS), lambda i: (0, 0)),
        ],
        out_specs=pl.BlockSpec((8, 128), lambda i: (0, 0)),
        out_shape=jax.ShapeDtypeStruct((8, 128), jnp.float32),
    )(cov6.T, xt)
    simple_inlet = out8[0:1, 0:6]

    rb = 8192
    x_sup, y_sup = pl.pallas_call(
        _sup_body,
        grid=(pl.cdiv(n_int, rb),),
        in_specs=[
            pl.BlockSpec((rb, 8), lambda i: (i, 0)),
            pl.BlockSpec((rb, 4), lambda i: (i, 0)),
        ],
        out_specs=[
            pl.BlockSpec((rb, 4), lambda i: (i, 0)),
            pl.BlockSpec((rb, 5), lambda i: (i, 0)),
        ],
        out_shape=[
            jax.ShapeDtypeStruct((n_int, 4), jnp.float32),
            jax.ShapeDtypeStruct((n_int, 5), jnp.float32),
        ],
    )(array_internal, array_sdf)

    X_sup = x_sup[None]
    Y_sup = y_sup[None]
    X_inlet = x_inlet[None].astype(jnp.float32)
    Simple_inlet = simple_inlet
    return (X_sup, Y_sup, X_inlet, Simple_inlet)
